# Initial kernel scaffold; baseline (speedup 1.0000x reference)
import jax
import jax.numpy as jnp
from jax.experimental import pallas as pl


def kernel(x, Wq, Wout, keys, values, scale):
    raise NotImplementedError("write your pallas kernel here")


# speedup vs baseline: 36.1287x; 36.1287x over previous
import functools
import math

import jax
import jax.numpy as jnp
from jax import lax
from jax.experimental import pallas as pl
from jax.experimental.pallas import tpu as pltpu
from jax.experimental.pallas import tpu_sc as plsc

S = 2048
MODEL_DIM = 1024
CACHE_SIZE = 65536
CACHE_DIM = 64
TOPK = 32

RB = 64
NCHUNK = 32
CW = CACHE_SIZE // NCHUNK
NCLS = 128
UB = CW // NCLS

_NEG = -1e30


def _extract_topk(f, k, need_pos):
    r, n = f.shape
    iota = lax.broadcasted_iota(jnp.int32, (r, n), 1)
    vals, poss = [], []
    for _ in range(k):
        m = jnp.max(f, axis=-1, keepdims=True)
        p = jnp.min(jnp.where(f == m, iota, n), axis=-1, keepdims=True)
        vals.append(m)
        poss.append(p)
        f = jnp.where(iota == p, _NEG, f)
    vals = jnp.concatenate(vals, axis=-1)
    if not need_pos:
        return vals, None
    return vals, jnp.concatenate(poss, axis=-1)


def _k1_body(x_ref, wq_ref, keys_ref, w_ref, col_ref, lg_ref):
    xb = x_ref[0]
    eps = jnp.float32(jnp.finfo(jnp.float32).eps)
    ms = jnp.mean(xb * xb, axis=-1, keepdims=True)
    xn = xb * lax.rsqrt(ms + eps)
    q = lax.dot_general(xn, wq_ref[...], (((1,), (1,)), ((), ())),
                        preferred_element_type=jnp.float32)
    q = q * jnp.float32(1.0 / math.sqrt(CACHE_DIM))

    m2 = jnp.full((RB, NCLS), _NEG, jnp.float32)
    for c in range(NCHUNK):
        kc = keys_ref[pl.ds(c * CW, CW), :]
        lg = lax.dot_general(q, kc, (((1,), (1,)), ((), ())),
                             preferred_element_type=jnp.float32)
        lg = lg.reshape(RB, UB, NCLS)
        lg_ref[c] = lg
        m2 = jnp.maximum(m2, jnp.max(lg, axis=1))

    _, cls = _extract_topk(m2, TOPK, True)

    cls_b = jnp.broadcast_to(cls[:, None, :], (RB, UB, TOPK))
    pieces = []
    for c in range(NCHUNK):
        pc = jnp.take_along_axis(lg_ref[c], cls_b, axis=-1)
        pieces.append(pc[:, None, :, :])
    c5 = jnp.concatenate(pieces, axis=1)

    m1 = jnp.max(c5, axis=2).reshape(RB, NCHUNK * TOPK)
    _, e = _extract_topk(m1, TOPK, True)

    idx_s = jnp.broadcast_to((e & 31)[:, None, None, :], (RB, NCHUNK, UB, TOPK))
    g1 = jnp.take_along_axis(c5, idx_s, axis=-1)
    g1 = g1.reshape(RB, NCHUNK, UB * TOPK)
    idx_g = jnp.broadcast_to((e >> 5)[:, None, :], (RB, UB, TOPK))
    idx_g = idx_g.reshape(RB, 1, UB * TOPK)
    f = None
    for b in range(NCHUNK // 8):
        sub = g1[:, b * 8:(b + 1) * 8, :]
        loc = jnp.clip(idx_g - b * 8, 0, 7)
        gb = jnp.take_along_axis(sub, loc, axis=1)
        f = gb if f is None else jnp.where((idx_g >> 3) == b, gb, f)
    f = f.reshape(RB, UB * TOPK)

    v, p = _extract_topk(f, TOPK, True)

    jj = p & 31
    ee = jnp.take_along_axis(e, jj, axis=-1)
    lsel = jnp.take_along_axis(cls, ee & 31, axis=-1)
    col = ((ee >> 5) * CW + (p >> 5) * NCLS + lsel)

    w = jnp.exp(v - v[:, :1])
    w = w / jnp.sum(w, axis=-1, keepdims=True)

    w_ref[...] = w
    col_ref[...] = col


def _k3_body(w_ref, g_ref, wout_ref, scale_ref, o_ref):
    w = w_ref[...]
    g = g_ref[...]
    ret = jnp.sum(w[:, :, None] * g, axis=1)
    out = lax.dot_general(ret, wout_ref[...], (((1,), (1,)), ((), ())),
                          preferred_element_type=jnp.float32)
    o_ref[0] = out * scale_ref[0]


def _make_k2():
    nw = 32
    chunk = 1024
    per_w = (S * TOPK) // nw
    mesh = plsc.VectorSubcoreMesh(core_axis_name="c", subcore_axis_name="s")

    @functools.partial(
        pl.kernel, mesh=mesh,
        compiler_params=pltpu.CompilerParams(use_tc_tiling_on_sc=False),
        out_type=jax.ShapeDtypeStruct((S * TOPK, CACHE_DIM), jnp.float32),
        scratch_types=[
            pltpu.VMEM((chunk,), jnp.int32),
            pltpu.VMEM((chunk, CACHE_DIM), jnp.float32),
            pltpu.SemaphoreType.DMA,
        ],
    )
    def k2(values_hbm, col_hbm, out_hbm, idx_v, rows_v, sem):
        wid = lax.axis_index("s") * 2 + lax.axis_index("c")
        for t in range(per_w // chunk):
            base = wid * per_w + t * chunk
            pltpu.sync_copy(col_hbm.at[pl.ds(base, chunk)], idx_v)
            pltpu.async_copy(values_hbm.at[idx_v], rows_v, sem).wait()
            pltpu.sync_copy(rows_v, out_hbm.at[pl.ds(base, chunk)])

    return k2


def kernel(x, Wq, Wout, keys, values, scale):
    nblk = S // RB
    w, col = pl.pallas_call(
        _k1_body,
        grid=(nblk,),
        in_specs=[
            pl.BlockSpec((1, RB, MODEL_DIM), lambda i: (0, i, 0)),
            pl.BlockSpec((CACHE_DIM, MODEL_DIM), lambda i: (0, 0)),
            pl.BlockSpec((CACHE_SIZE, CACHE_DIM), lambda i: (0, 0)),
        ],
        out_specs=[
            pl.BlockSpec((RB, TOPK), lambda i: (i, 0)),
            pl.BlockSpec((RB, TOPK), lambda i: (i, 0)),
        ],
        out_shape=[
            jax.ShapeDtypeStruct((S, TOPK), jnp.float32),
            jax.ShapeDtypeStruct((S, TOPK), jnp.int32),
        ],
        scratch_shapes=[pltpu.VMEM((NCHUNK, RB, UB, NCLS), jnp.float32)],
    )(x, Wq, keys)

    gathered = _make_k2()(values, col.reshape(S * TOPK))
    gathered = gathered.reshape(S, TOPK, CACHE_DIM)

    rk = 256
    out = pl.pallas_call(
        _k3_body,
        grid=(S // rk,),
        in_specs=[
            pl.BlockSpec((rk, TOPK), lambda i: (i, 0)),
            pl.BlockSpec((rk, TOPK, CACHE_DIM), lambda i: (i, 0, 0)),
            pl.BlockSpec((MODEL_DIM, CACHE_DIM), lambda i: (0, 0)),
            pl.BlockSpec(memory_space=pltpu.SMEM),
        ],
        out_specs=pl.BlockSpec((1, rk, MODEL_DIM), lambda i: (0, i, 0)),
        out_shape=jax.ShapeDtypeStruct((1, S, MODEL_DIM), jnp.float32),
    )(w, gathered, Wout, scale.reshape(1))
    return out
